# baseline (speedup 1.0000x reference)
import functools

import jax
import jax.numpy as jnp
from jax import lax
from jax.experimental import pallas as pl
from jax.experimental.pallas import tpu as pltpu
from jax.experimental.pallas import tpu_sc as plsc

NC = 2
NS = 16
NW = NC * NS
CHUNK = 128
NBUF = 8
ROWBLK = 512
L = 16


def _tc_project(pos, vel, W_rel, W_root, W_pred, b_rel, b_pred, npad):
  n, d = pos.shape
  emb = 2 * d
  out_w = W_pred.shape[1]
  grid = (npad + ROWBLK - 1) // ROWBLK

  def body(pos_ref, vel_ref, wrel_ref, wroot_ref, wpred_ref, brel_ref,
           bpred_ref, tab_ref, z_ref, cw):
    wp16 = jnp.concatenate(
        [wpred_ref[...], jnp.zeros((emb, L - out_w), jnp.float32)], axis=1)

    @pl.when(pl.program_id(0) == 0)
    def _():
      cw[:, :L] = jnp.dot(wrel_ref[...], wp16,
                          preferred_element_type=jnp.float32)
      cw[:, L:] = jnp.dot(wroot_ref[...], wp16,
                          preferred_element_type=jnp.float32)

    biasT = lax.dot_general(wp16, brel_ref[...], (((0,), (0,)), ((), ())),
                            preferred_element_type=jnp.float32)
    biasT = biasT + jnp.concatenate(
        [bpred_ref[...], jnp.zeros((L - out_w, 1), jnp.float32)], axis=0)
    hv = jnp.concatenate([pos_ref[...], vel_ref[...]], axis=1)
    yzT = lax.dot_general(cw[...], hv, (((0,), (1,)), ((), ())),
                          preferred_element_type=jnp.float32)
    tab_ref[...] = yzT[:L, :]
    z_ref[...] = yzT[L:, :] + biasT

  return pl.pallas_call(
      body,
      grid=(grid,),
      in_specs=[
          pl.BlockSpec((ROWBLK, d), lambda i: (i, 0)),
          pl.BlockSpec((ROWBLK, d), lambda i: (i, 0)),
          pl.BlockSpec((emb, emb), lambda i: (0, 0)),
          pl.BlockSpec((emb, emb), lambda i: (0, 0)),
          pl.BlockSpec((emb, out_w), lambda i: (0, 0)),
          pl.BlockSpec((emb, 1), lambda i: (0, 0)),
          pl.BlockSpec((out_w, 1), lambda i: (0, 0)),
      ],
      out_specs=[
          pl.BlockSpec((L, ROWBLK), lambda i: (0, i)),
          pl.BlockSpec((L, ROWBLK), lambda i: (0, i)),
      ],
      out_shape=[
          jax.ShapeDtypeStruct((L, npad), jnp.float32),
          jax.ShapeDtypeStruct((L, npad), jnp.float32),
      ],
      scratch_shapes=[
          pltpu.VMEM((emb, 2 * L), jnp.float32),
      ],
  )(pos, vel, W_rel, W_root, W_pred, b_rel.reshape(emb, 1),
    b_pred.reshape(out_w, 1))


def _sc_segment_sum(tab, z16, ei2, npad, n):
  nch = ei2.shape[0] // 2
  q, rem = divmod(nch, NW)
  cpt = -(-(q + (1 if rem else 0)) // NBUF) * NBUF
  rows_pt = npad // NS
  n_dummy = npad - n
  nrounds = cpt // NBUF
  mesh = plsc.VectorSubcoreMesh(core_axis_name="c", subcore_axis_name="s")

  @functools.partial(
      pl.kernel,
      mesh=mesh,
      out_type=jax.ShapeDtypeStruct((NC, npad, L), jnp.float32),
      compiler_params=pltpu.CompilerParams(use_tc_tiling_on_sc=False),
      scratch_types=[
          pltpu.VMEM((cpt, CHUNK), jnp.int32),
          pltpu.VMEM((cpt, CHUNK), jnp.int32),
          [pltpu.VMEM((CHUNK, L), jnp.float32)] * NBUF,
          pltpu.VMEM((rows_pt, L), jnp.float32),
          pltpu.VMEM_SHARED((npad, L), jnp.float32),
          [pltpu.SemaphoreType.DMA] * NBUF,
          [pltpu.SemaphoreType.DMA] * NBUF,
          pltpu.SemaphoreType.DMA,
      ],
  )
  def sck(tab_hbm, z_hbm, ei_hbm, out_hbm,
          idx_s, idx_d, vals, buf, acc_sh, gsem, ssem, isem):
    c = lax.axis_index("c")
    s = lax.axis_index("s")
    w = c * NS + s
    r0 = s * rows_pt
    rows = pl.ds(r0, rows_pt)
    ncw = q + jnp.where(w < rem, 1, 0)
    c0 = w * q + jnp.minimum(w, rem)

    def stage(r, carry):
      pltpu.async_copy(ei_hbm.at[c0 + r], idx_s.at[r], isem)
      pltpu.async_copy(ei_hbm.at[nch + c0 + r], idx_d.at[r], isem)
      return carry

    lax.fori_loop(0, ncw, stage, 0)

    iota = lax.iota(jnp.int32, L)
    base = iota + CHUNK * s

    def fill(r, carry):
      for col0 in range(0, CHUNK, L):
        k = r * (CHUNK // L) + col0 // L
        idx_s[r, pl.ds(col0, L)] = (base + L * k) % n
        idx_d[r, pl.ds(col0, L)] = n + (base + 7 * k) % n_dummy
      return carry

    lax.fori_loop(ncw, cpt, fill, 0)

    @pl.when(c == 0)
    def _():
      pltpu.sync_copy(z_hbm.at[rows], buf)

    @pl.when(c != 0)
    def _():
      zv = jnp.zeros((L,), jnp.float32)

      def zbody(r, carry):
        buf[r] = zv
        return carry

      lax.fori_loop(0, rows_pt, zbody, 0)

    pltpu.sync_copy(buf, acc_sh.at[rows])

    def drain(r, carry):
      pltpu.make_async_copy(ei_hbm.at[0], idx_s.at[0], isem).wait()
      pltpu.make_async_copy(ei_hbm.at[0], idx_d.at[0], isem).wait()
      return carry

    lax.fori_loop(0, ncw, drain, 0)
    plsc.subcore_barrier()

    for b in range(NBUF):
      pltpu.async_copy(tab_hbm.at[idx_s.at[b]], vals[b], gsem[b])

    def round_body(g, carry):
      scats = []
      for b in range(NBUF):
        pltpu.make_async_copy(tab_hbm.at[pl.ds(0, CHUNK)], vals[b],
                              gsem[b]).wait()
        scats.append(
            pltpu.async_copy(vals[b], acc_sh.at[idx_d.at[g * NBUF + b]],
                             ssem[b], add=True))
      for b in range(NBUF):
        scats[b].wait()
        jn = (g + 1) * NBUF + b

        @pl.when(jn < cpt)
        def _():
          pltpu.async_copy(tab_hbm.at[idx_s.at[jn]], vals[b], gsem[b])

      return carry

    lax.fori_loop(0, nrounds, round_body, 0)
    plsc.subcore_barrier()

    pltpu.sync_copy(acc_sh.at[rows], buf)
    pltpu.sync_copy(buf, out_hbm.at[c, rows])

  return sck(tab, z16, ei2)


def _tc_combine(pT, n, out_w):
  npad = pT.shape[2]
  grid = (npad + ROWBLK - 1) // ROWBLK

  def body(p_ref, out_ref):
    acc = p_ref[0] + p_ref[1]
    out_ref[...] = acc[:out_w, :]

  return pl.pallas_call(
      body,
      grid=(grid,),
      in_specs=[pl.BlockSpec((NC, L, ROWBLK), lambda i: (0, 0, i))],
      out_specs=pl.BlockSpec((out_w, ROWBLK), lambda i: (0, i)),
      out_shape=jax.ShapeDtypeStruct((out_w, n), jnp.float32),
  )(pT)


def kernel(pos, vel, edge_index, W_rel, b_rel, W_root, W_pred, b_pred):
  n, d = pos.shape
  e = edge_index.shape[1]
  out_w = W_pred.shape[1]
  assert e % CHUNK == 0

  rows_pt = -(-(n + 64) // (NS * 8)) * 8
  npad = NS * rows_pt

  tabT, zT = _tc_project(pos, vel, W_rel, W_root, W_pred, b_rel, b_pred, npad)
  tab = tabT.T
  z16 = zT.T
  ei2 = edge_index.reshape(2 * e // CHUNK, CHUNK)
  partials = _sc_segment_sum(tab, z16, ei2, npad, n)
  pT = jnp.transpose(partials, (0, 2, 1))
  return _tc_combine(pT, n, out_w).T
